# Initial kernel scaffold; baseline (speedup 1.0000x reference)
import jax
import jax.numpy as jnp
from jax.experimental import pallas as pl


def kernel(memory_state, entity_emb, relation_emb, W_tok, b_tok):
    raise NotImplementedError("write your pallas kernel here")


# speedup vs baseline: 2.3694x; 2.3694x over previous
import functools

import jax
import jax.numpy as jnp
from jax import lax
from jax.experimental import pallas as pl
from jax.experimental.pallas import tpu as pltpu
from jax.experimental.pallas import tpu_sc as plsc

S = 16384
E = 64
NUM_ROWS = 1000

NC = 2
NS = 16
NW = NC * NS
TOK_PER_W = S // NW
CHUNK = 128
NCH = TOK_PER_W // CHUNK


def _project_body(ent_ref, rel_ref, w_ref, b_ref, th_ref, tr_ref, tt_ref):
    ent = ent_ref[...]
    rel = rel_ref[...]
    w = w_ref[...]
    dn = (((1,), (1,)), ((), ()))
    th_ref[...] = lax.dot_general(ent, w[:, 0:E], dn,
                                  preferred_element_type=jnp.float32)
    tr_ref[...] = lax.dot_general(rel, w[:, E:2 * E], dn,
                                  preferred_element_type=jnp.float32) + b_ref[...]
    tt_ref[...] = lax.dot_general(ent, w[:, 2 * E:3 * E], dn,
                                  preferred_element_type=jnp.float32)


def _project_tables(entity_emb, relation_emb, W_tok, b_tok):
    out_shape = [jax.ShapeDtypeStruct((NUM_ROWS, E), jnp.float32)] * 3
    return pl.pallas_call(_project_body, out_shape=out_shape)(
        entity_emb, relation_emb, W_tok, b_tok.reshape(1, E))


def _sc_body(h_hbm, r_hbm, t_hbm, th_hbm, tr_hbm, tt_hbm, out_hbm,
             ih, ir, it, gh, gr, gt, sem):
    wid = lax.axis_index("s") * NC + lax.axis_index("c")
    base = wid * TOK_PER_W

    def chunk_body(ci, carry):
        cb = base + ci * CHUNK
        pltpu.sync_copy(h_hbm.at[pl.ds(cb, CHUNK)], ih)
        pltpu.sync_copy(r_hbm.at[pl.ds(cb, CHUNK)], ir)
        pltpu.sync_copy(t_hbm.at[pl.ds(cb, CHUNK)], it)
        ch = pltpu.async_copy(th_hbm.at[ih], gh, sem)
        cr = pltpu.async_copy(tr_hbm.at[ir], gr, sem)
        ct = pltpu.async_copy(tt_hbm.at[it], gt, sem)
        ch.wait()
        cr.wait()
        ct.wait()

        def row_body(i, c2):
            for j in range(E // 16):
                sl = pl.ds(j * 16, 16)
                gh[i, sl] = gh[i, sl] + gr[i, sl] + gt[i, sl]
            return c2

        lax.fori_loop(0, CHUNK, row_body, 0, unroll=2)
        pltpu.sync_copy(gh, out_hbm.at[pl.ds(cb, CHUNK)])
        return carry

    lax.fori_loop(0, NCH, chunk_body, 0)


_sc_gather = functools.partial(
    pl.kernel,
    out_type=jax.ShapeDtypeStruct((S, E), jnp.float32),
    mesh=plsc.VectorSubcoreMesh(core_axis_name="c", subcore_axis_name="s"),
    scratch_types=[
        pltpu.VMEM((CHUNK,), jnp.int32),
        pltpu.VMEM((CHUNK,), jnp.int32),
        pltpu.VMEM((CHUNK,), jnp.int32),
        pltpu.VMEM((CHUNK, E), jnp.float32),
        pltpu.VMEM((CHUNK, E), jnp.float32),
        pltpu.VMEM((CHUNK, E), jnp.float32),
        pltpu.SemaphoreType.DMA,
    ],
    compiler_params=pltpu.CompilerParams(use_tc_tiling_on_sc=False),
)(_sc_body)


def kernel(memory_state, entity_emb, relation_emb, W_tok, b_tok):
    h = memory_state[:, 0]
    r = memory_state[:, 1]
    t = memory_state[:, 2]
    tbl_h, tbl_r, tbl_t = _project_tables(entity_emb, relation_emb, W_tok, b_tok)
    return _sc_gather(h, r, t, tbl_h, tbl_r, tbl_t)


# speedup vs baseline: 2.4804x; 1.0468x over previous
import functools

import jax
import jax.numpy as jnp
from jax import lax
from jax.experimental import pallas as pl
from jax.experimental.pallas import tpu as pltpu
from jax.experimental.pallas import tpu_sc as plsc

S = 16384
E = 64
NUM_ROWS = 1000

NC = 2
NS = 16
NW = NC * NS
TOK_PER_W = S // NW
CHUNK = 128
NCH = TOK_PER_W // CHUNK


def _project_body(ent_ref, rel_ref, w_ref, b_ref, th_ref, tr_ref, tt_ref):
    ent = ent_ref[...]
    rel = rel_ref[...]
    w = w_ref[...]
    dn = (((1,), (1,)), ((), ()))
    th_ref[...] = lax.dot_general(ent, w[:, 0:E], dn,
                                  preferred_element_type=jnp.float32)
    tr_ref[...] = lax.dot_general(rel, w[:, E:2 * E], dn,
                                  preferred_element_type=jnp.float32) + b_ref[...]
    tt_ref[...] = lax.dot_general(ent, w[:, 2 * E:3 * E], dn,
                                  preferred_element_type=jnp.float32)


def _project_tables(entity_emb, relation_emb, W_tok, b_tok):
    out_shape = [jax.ShapeDtypeStruct((NUM_ROWS, E), jnp.float32)] * 3
    return pl.pallas_call(_project_body, out_shape=out_shape)(
        entity_emb, relation_emb, W_tok, b_tok.reshape(1, E))


def _sc_body(ms_hbm, th_hbm, tr_hbm, tt_hbm, out_hbm,
             msb, ih, ir, it, gh, gr, gt, acc, semg0, semg1, semw):
    wid = lax.axis_index("s") * NC + lax.axis_index("c")
    base = pl.multiple_of(wid * TOK_PER_W, TOK_PER_W)

    pltpu.sync_copy(ms_hbm.at[pl.ds(base * 3, TOK_PER_W * 3)], msb)

    lane3 = lax.iota(jnp.int32, 16) * 3
    for g in range(TOK_PER_W // 16):
        src = lane3 + g * 48
        c, pos = g // (CHUNK // 16), (g % (CHUNK // 16)) * 16
        ih[c, pl.ds(pos, 16)] = plsc.load_gather(msb, [src])
        ir[c, pl.ds(pos, 16)] = plsc.load_gather(msb, [src + 1])
        it[c, pl.ds(pos, 16)] = plsc.load_gather(msb, [src + 2])

    def start_gathers(c):
        b = c % 2
        sem = semg0 if b == 0 else semg1
        return (pltpu.async_copy(th_hbm.at[ih.at[c]], gh.at[b], sem),
                pltpu.async_copy(tr_hbm.at[ir.at[c]], gr.at[b], sem),
                pltpu.async_copy(tt_hbm.at[it.at[c]], gt.at[b], sem))

    def compute(c):
        b = c % 2

        def body(i, carry):
            for j in range(E // 16):
                sl = pl.ds(j * 16, 16)
                acc[b, pl.ds(i * E + j * 16, 16)] = (
                    gh[b, i, sl] + gr[b, i, sl] + gt[b, i, sl])
            return carry

        lax.fori_loop(0, CHUNK, body, 0)

    gcur = start_gathers(0)
    wbs = {}
    for c in range(NCH):
        b = c % 2
        if c + 1 < NCH:
            gnext = start_gathers(c + 1)
        for d in gcur:
            d.wait()
        if c - 2 >= 0:
            wbs.pop(c - 2).wait()
        compute(c)
        wbs[c] = pltpu.async_copy(
            acc.at[b],
            out_hbm.at[pl.ds((base + c * CHUNK) * E, CHUNK * E)], semw)
        if c + 1 < NCH:
            gcur = gnext
    for c in sorted(wbs):
        wbs[c].wait()


_sc_gather = functools.partial(
    pl.kernel,
    out_type=jax.ShapeDtypeStruct((S * E,), jnp.float32),
    mesh=plsc.VectorSubcoreMesh(core_axis_name="c", subcore_axis_name="s"),
    scratch_types=[
        pltpu.VMEM((TOK_PER_W * 3,), jnp.int32),
        pltpu.VMEM((NCH, CHUNK), jnp.int32),
        pltpu.VMEM((NCH, CHUNK), jnp.int32),
        pltpu.VMEM((NCH, CHUNK), jnp.int32),
        pltpu.VMEM((2, CHUNK, E), jnp.float32),
        pltpu.VMEM((2, CHUNK, E), jnp.float32),
        pltpu.VMEM((2, CHUNK, E), jnp.float32),
        pltpu.VMEM((2, CHUNK * E), jnp.float32),
        pltpu.SemaphoreType.DMA,
        pltpu.SemaphoreType.DMA,
        pltpu.SemaphoreType.DMA,
    ],
    compiler_params=pltpu.CompilerParams(use_tc_tiling_on_sc=False,
                                         needs_layout_passes=False),
)(_sc_body)


def kernel(memory_state, entity_emb, relation_emb, W_tok, b_tok):
    tbl_h, tbl_r, tbl_t = _project_tables(entity_emb, relation_emb, W_tok, b_tok)
    flat = _sc_gather(memory_state.reshape(-1), tbl_h, tbl_r, tbl_t)
    return flat.reshape(S, E)
